# Initial kernel scaffold; baseline (speedup 1.0000x reference)
import jax
import jax.numpy as jnp
from jax.experimental import pallas as pl


def kernel(x_index, sequences, embed, weight, weight_proj, W_ih, W_hh, b_ih, b_hh, out_W, out_b):
    raise NotImplementedError("write your pallas kernel here")


# speedup vs baseline: 16.2704x; 16.2704x over previous
import functools

import jax
import jax.numpy as jnp
from jax.experimental import pallas as pl
from jax.experimental.pallas import tpu as pltpu


def _scan_body(seq_ref, node_emb_ref, w_ih_t_ref, b_ih_ref, w_hh_t_ref,
               b_hh_ref, weight_ref, wp_ref, out_w_t_ref, out_b_ref,
               out_ref, h_ref):
    n_nodes, hid = h_ref.shape

    h_ref[...] = jnp.zeros_like(h_ref)

    def step(i, _):
        nid = seq_ref[0, i]
        parent = seq_ref[1, i]
        prior = seq_ref[2, i]
        topo = h_ref[pl.ds(parent, 1), :]
        temp = h_ref[pl.ds(prior, 1), :]
        x = node_emb_ref[pl.ds(nid, 1), :]
        gi = jnp.dot(x, w_ih_t_ref[...],
                     preferred_element_type=jnp.float32) + b_ih_ref[...]
        gh = jnp.dot(temp, w_hh_t_ref[...],
                     preferred_element_type=jnp.float32) + b_hh_ref[...]
        i_r, i_z, i_n = (gi[:, :hid], gi[:, hid:2 * hid], gi[:, 2 * hid:])
        h_r, h_z, h_n = (gh[:, :hid], gh[:, hid:2 * hid], gh[:, 2 * hid:])
        r = jax.nn.sigmoid(i_r + h_r)
        z = jax.nn.sigmoid(i_z + h_z)
        n = jnp.tanh(i_n + r * h_n)
        h1 = (1.0 - z) * n + z * temp
        u0 = jnp.tanh(jnp.dot(topo, weight_ref[...],
                              preferred_element_type=jnp.float32))
        u1 = jnp.tanh(jnp.dot(h1, weight_ref[...],
                              preferred_element_type=jnp.float32))
        a0 = jnp.sum(u0 * wp_ref[...])
        a1 = jnp.sum(u1 * wp_ref[...])
        m = jnp.maximum(a0, a1)
        e0 = jnp.exp(a0 - m)
        e1 = jnp.exp(a1 - m)
        h_new = (topo * e0 + h1 * e1) / (e0 + e1)
        h_ref[pl.ds(nid, 1), :] = h_new
        return 0

    jax.lax.fori_loop(0, n_nodes, step, 0)

    h_last = h_ref[pl.ds(n_nodes - 1, 1), :]
    logits = jnp.dot(h_last, out_w_t_ref[...],
                     preferred_element_type=jnp.float32) + out_b_ref[...]
    lm = logits - jnp.max(logits)
    out_ref[...] = lm - jnp.log(jnp.sum(jnp.exp(lm)))


def _rnn_scan(seqs, node_emb, W_ih, W_hh, b_ih, b_hh, weight, weight_proj,
              out_W, out_b, *, interpret=False):
    n_nodes, hid = node_emb.shape[0], weight.shape[0]
    nclass = out_W.shape[0]
    grid_spec = pltpu.PrefetchScalarGridSpec(
        num_scalar_prefetch=1,
        grid=(1,),
        in_specs=[
            pl.BlockSpec(node_emb.shape, lambda i, s: (0, 0)),
            pl.BlockSpec((hid, 3 * hid), lambda i, s: (0, 0)),
            pl.BlockSpec((1, 3 * hid), lambda i, s: (0, 0)),
            pl.BlockSpec((hid, 3 * hid), lambda i, s: (0, 0)),
            pl.BlockSpec((1, 3 * hid), lambda i, s: (0, 0)),
            pl.BlockSpec((hid, hid), lambda i, s: (0, 0)),
            pl.BlockSpec((1, hid), lambda i, s: (0, 0)),
            pl.BlockSpec((hid, nclass), lambda i, s: (0, 0)),
            pl.BlockSpec((1, nclass), lambda i, s: (0, 0)),
        ],
        out_specs=pl.BlockSpec((1, nclass), lambda i, s: (0, 0)),
        scratch_shapes=[pltpu.VMEM((n_nodes, hid), jnp.float32)],
    )
    return pl.pallas_call(
        _scan_body,
        grid_spec=grid_spec,
        out_shape=jax.ShapeDtypeStruct((1, nclass), jnp.float32),
        interpret=interpret,
    )(seqs, node_emb, W_ih.T, b_ih.reshape(1, -1), W_hh.T,
      b_hh.reshape(1, -1), weight, weight_proj.reshape(1, -1),
      out_W.T, out_b.reshape(1, -1))


def kernel(x_index, sequences, embed, weight, weight_proj, W_ih, W_hh, b_ih,
           b_hh, out_W, out_b):
    node_emb = jnp.take(embed, x_index, axis=0).sum(axis=1) / x_index.shape[1]
    seqs = sequences[:, :, 0].T
    return _rnn_scan(seqs, node_emb, W_ih, W_hh, b_ih, b_hh, weight,
                     weight_proj, out_W, out_b)


# speedup vs baseline: 16.4701x; 1.0123x over previous
import functools

import jax
import jax.numpy as jnp
from jax import lax
from jax.experimental import pallas as pl
from jax.experimental.pallas import tpu as pltpu
from jax.experimental.pallas import tpu_sc as plsc

_SC_INFO = plsc.get_sparse_core_info()
_NC, _NS, _L = _SC_INFO.num_cores, _SC_INFO.num_subcores, _SC_INFO.num_lanes
_NW = _NC * _NS

_WRD = 16
_CHN = 8
_ROWS_PER_CHUNK = _CHN * _WRD


def _emb_body(idx_hbm, embed_hbm, out_hbm, idx_v, rows_v, out_v, sem):
    npw = out_v.shape[0]
    nchunks = npw // _CHN
    d = embed_hbm.shape[1]
    nblk = d // _L
    wid = lax.axis_index("s") * _NC + lax.axis_index("c")
    pltpu.sync_copy(idx_hbm.at[pl.ds(wid * nchunks, nchunks)], idx_v)

    def chunk(ci, _):
        pltpu.async_copy(embed_hbm.at[idx_v.at[ci]], rows_v, sem).wait()

        def node(j, _):
            for cb in range(nblk):
                acc = rows_v[j * _WRD, pl.ds(cb * _L, _L)]
                for r in range(1, _WRD):
                    acc = acc + rows_v[j * _WRD + r, pl.ds(cb * _L, _L)]
                out_v[ci * _CHN + j, pl.ds(cb * _L, _L)] = acc * (1.0 / _WRD)
            return 0

        lax.fori_loop(0, _CHN, node, 0)
        return 0

    lax.fori_loop(0, nchunks, chunk, 0)
    pltpu.sync_copy(out_v, out_hbm.at[pl.ds(wid * npw, npw)])


def _embedding_mean(x_index, embed):
    n, wrd = x_index.shape
    d = embed.shape[1]
    npw = -(-n // (_NW * _CHN)) * _CHN
    b = npw * _NW
    idx = jnp.pad(x_index, ((0, b - n), (0, 0))).reshape(-1, _ROWS_PER_CHUNK)
    mesh = plsc.VectorSubcoreMesh(core_axis_name="c", subcore_axis_name="s")
    emb_k = functools.partial(
        pl.kernel,
        mesh=mesh,
        out_type=jax.ShapeDtypeStruct((b, d), jnp.float32),
        scratch_types=[
            pltpu.VMEM((npw // _CHN, _ROWS_PER_CHUNK), jnp.int32),
            pltpu.VMEM((_ROWS_PER_CHUNK, d), jnp.float32),
            pltpu.VMEM((npw, d), jnp.float32),
            pltpu.SemaphoreType.DMA,
        ],
    )(_emb_body)
    return emb_k(idx, embed)


def _scan_body(seq_ref, node_emb_ref, w_ih_t_ref, b_ih_ref, w_hh_t_ref,
               b_hh_ref, weight_ref, wp_ref, out_w_t_ref, out_b_ref,
               out_ref, h_ref):
    n_nodes, hid = h_ref.shape
    n_steps = seq_ref.shape[1]

    h_ref[...] = jnp.zeros_like(h_ref)

    def step(i, _):
        nid = seq_ref[0, i]
        parent = seq_ref[1, i]
        prior = seq_ref[2, i]
        topo = h_ref[pl.ds(parent, 1), :]
        temp = h_ref[pl.ds(prior, 1), :]
        x = node_emb_ref[pl.ds(nid, 1), :]
        gi = jnp.dot(x, w_ih_t_ref[...],
                     preferred_element_type=jnp.float32) + b_ih_ref[...]
        gh = jnp.dot(temp, w_hh_t_ref[...],
                     preferred_element_type=jnp.float32) + b_hh_ref[...]
        i_r, i_z, i_n = (gi[:, :hid], gi[:, hid:2 * hid], gi[:, 2 * hid:])
        h_r, h_z, h_n = (gh[:, :hid], gh[:, hid:2 * hid], gh[:, 2 * hid:])
        r = jax.nn.sigmoid(i_r + h_r)
        z = jax.nn.sigmoid(i_z + h_z)
        n = jnp.tanh(i_n + r * h_n)
        h1 = (1.0 - z) * n + z * temp
        u0 = jnp.tanh(jnp.dot(topo, weight_ref[...],
                              preferred_element_type=jnp.float32))
        u1 = jnp.tanh(jnp.dot(h1, weight_ref[...],
                              preferred_element_type=jnp.float32))
        a0 = jnp.sum(u0 * wp_ref[...])
        a1 = jnp.sum(u1 * wp_ref[...])
        m = jnp.maximum(a0, a1)
        e0 = jnp.exp(a0 - m)
        e1 = jnp.exp(a1 - m)
        h_new = (topo * e0 + h1 * e1) / (e0 + e1)
        h_ref[pl.ds(nid, 1), :] = h_new
        return 0

    jax.lax.fori_loop(0, n_steps, step, 0)

    h_last = h_ref[pl.ds(n_nodes - 1, 1), :]
    logits = jnp.dot(h_last, out_w_t_ref[...],
                     preferred_element_type=jnp.float32) + out_b_ref[...]
    lm = logits - jnp.max(logits)
    out_ref[...] = lm - jnp.log(jnp.sum(jnp.exp(lm)))


def _rnn_scan(seqs, node_emb, W_ih, W_hh, b_ih, b_hh, weight, weight_proj,
              out_W, out_b, *, interpret=False):
    n_nodes, hid = seqs.shape[1], weight.shape[0]
    nclass = out_W.shape[0]
    grid_spec = pltpu.PrefetchScalarGridSpec(
        num_scalar_prefetch=1,
        grid=(1,),
        in_specs=[
            pl.BlockSpec(node_emb.shape, lambda i, s: (0, 0)),
            pl.BlockSpec((hid, 3 * hid), lambda i, s: (0, 0)),
            pl.BlockSpec((1, 3 * hid), lambda i, s: (0, 0)),
            pl.BlockSpec((hid, 3 * hid), lambda i, s: (0, 0)),
            pl.BlockSpec((1, 3 * hid), lambda i, s: (0, 0)),
            pl.BlockSpec((hid, hid), lambda i, s: (0, 0)),
            pl.BlockSpec((1, hid), lambda i, s: (0, 0)),
            pl.BlockSpec((hid, nclass), lambda i, s: (0, 0)),
            pl.BlockSpec((1, nclass), lambda i, s: (0, 0)),
        ],
        out_specs=pl.BlockSpec((1, nclass), lambda i, s: (0, 0)),
        scratch_shapes=[pltpu.VMEM((n_nodes, hid), jnp.float32)],
    )
    return pl.pallas_call(
        _scan_body,
        grid_spec=grid_spec,
        out_shape=jax.ShapeDtypeStruct((1, nclass), jnp.float32),
        interpret=interpret,
    )(seqs, node_emb, W_ih.T, b_ih.reshape(1, -1), W_hh.T,
      b_hh.reshape(1, -1), weight, weight_proj.reshape(1, -1),
      out_W.T, out_b.reshape(1, -1))


def kernel(x_index, sequences, embed, weight, weight_proj, W_ih, W_hh, b_ih,
           b_hh, out_W, out_b):
    node_emb = _embedding_mean(x_index, embed)
    seqs = sequences[:, :, 0].T
    return _rnn_scan(seqs, node_emb, W_ih, W_hh, b_ih, b_hh, weight,
                     weight_proj, out_W, out_b)


# speedup vs baseline: 124.5540x; 7.5625x over previous
import functools

import jax
import jax.numpy as jnp
from jax import lax
from jax.experimental import pallas as pl
from jax.experimental.pallas import tpu as pltpu
from jax.experimental.pallas import tpu_sc as plsc

_NC, _NS, _L = 2, 16, 16
_NW = _NC * _NS

_WRD = 16
_CHN = 8
_ROWS_PER_CHUNK = _CHN * _WRD


def _emb_body(idx_hbm, embed_hbm, out_hbm, idx_v, rows_v, out_v, sem):
    npw = out_v.shape[0]
    nchunks = npw // _CHN
    d = embed_hbm.shape[1]
    nblk = d // _L
    wid = lax.axis_index("s") * _NC + lax.axis_index("c")
    pltpu.sync_copy(idx_hbm.at[pl.ds(wid * nchunks, nchunks)], idx_v)

    def chunk(ci, _):
        pltpu.async_copy(embed_hbm.at[idx_v.at[ci]], rows_v, sem).wait()

        def node(j, _):
            for cb in range(nblk):
                acc = rows_v[j * _WRD, pl.ds(cb * _L, _L)]
                for r in range(1, _WRD):
                    acc = acc + rows_v[j * _WRD + r, pl.ds(cb * _L, _L)]
                out_v[ci * _CHN + j, pl.ds(cb * _L, _L)] = acc * (1.0 / _WRD)
            return 0

        lax.fori_loop(0, _CHN, node, 0)
        return 0

    lax.fori_loop(0, nchunks, chunk, 0)
    pltpu.sync_copy(out_v, out_hbm.at[pl.ds(wid * npw, npw)])


def _embedding_mean(x_index, embed):
    n, wrd = x_index.shape
    d = embed.shape[1]
    npw = -(-n // (_NW * _CHN)) * _CHN
    b = npw * _NW
    idx = jnp.pad(x_index, ((0, b - n), (0, 0))).reshape(-1, _ROWS_PER_CHUNK)
    mesh = plsc.VectorSubcoreMesh(core_axis_name="c", subcore_axis_name="s")
    emb_k = functools.partial(
        pl.kernel,
        mesh=mesh,
        out_type=jax.ShapeDtypeStruct((b, d), jnp.float32),
        scratch_types=[
            pltpu.VMEM((npw // _CHN, _ROWS_PER_CHUNK), jnp.int32),
            pltpu.VMEM((_ROWS_PER_CHUNK, d), jnp.float32),
            pltpu.VMEM((npw, d), jnp.float32),
            pltpu.SemaphoreType.DMA,
        ],
    )(_emb_body)
    return emb_k(idx, embed)


_B = 16


def _scan_body(seq_ref, flag_ref, node_emb_ref, w_ih_t_ref, b_ih_ref,
               w_hh_t_ref, b_hh_ref, weight_ref, wp_ref, out_w_t_ref,
               out_b_ref, out_ref, h_ref):
    n_nodes, hid = h_ref.shape
    n_steps = seq_ref.shape[1]

    h_ref[...] = jnp.zeros_like(h_ref)

    def gru_att(x, temp, topo):
        gi = jnp.dot(x, w_ih_t_ref[...],
                     preferred_element_type=jnp.float32) + b_ih_ref[...]
        gh = jnp.dot(temp, w_hh_t_ref[...],
                     preferred_element_type=jnp.float32) + b_hh_ref[...]
        i_r, i_z, i_n = (gi[:, :hid], gi[:, hid:2 * hid], gi[:, 2 * hid:])
        h_r, h_z, h_n = (gh[:, :hid], gh[:, hid:2 * hid], gh[:, 2 * hid:])
        r = jax.nn.sigmoid(i_r + h_r)
        z = jax.nn.sigmoid(i_z + h_z)
        n = jnp.tanh(i_n + r * h_n)
        h1 = (1.0 - z) * n + z * temp
        u0 = jnp.tanh(jnp.dot(topo, weight_ref[...],
                              preferred_element_type=jnp.float32))
        u1 = jnp.tanh(jnp.dot(h1, weight_ref[...],
                              preferred_element_type=jnp.float32))
        a0 = jnp.sum(u0 * wp_ref[...], axis=1, keepdims=True)
        a1 = jnp.sum(u1 * wp_ref[...], axis=1, keepdims=True)
        s = jax.nn.sigmoid(a1 - a0)
        return topo + s * (h1 - topo)

    def step(i, _):
        nid = seq_ref[0, i]
        parent = seq_ref[1, i]
        prior = seq_ref[2, i]
        h_new = gru_att(node_emb_ref[pl.ds(nid, 1), :],
                        h_ref[pl.ds(prior, 1), :],
                        h_ref[pl.ds(parent, 1), :])
        h_ref[pl.ds(nid, 1), :] = h_new
        return 0

    def batch(bi, _):
        base = bi * _B
        flag = flag_ref[bi]

        @pl.when(flag == 0)
        def _fast():
            idxs = [(seq_ref[0, base + j], seq_ref[1, base + j],
                     seq_ref[2, base + j]) for j in range(_B)]
            x = jnp.concatenate(
                [node_emb_ref[pl.ds(t[0], 1), :] for t in idxs], axis=0)
            topo = jnp.concatenate(
                [h_ref[pl.ds(t[1], 1), :] for t in idxs], axis=0)
            temp = jnp.concatenate(
                [h_ref[pl.ds(t[2], 1), :] for t in idxs], axis=0)
            h_new = gru_att(x, temp, topo)
            for j in range(_B):
                h_ref[pl.ds(idxs[j][0], 1), :] = h_new[j:j + 1, :]

        @pl.when(flag != 0)
        def _slow():
            jax.lax.fori_loop(base, base + _B, step, 0)

        return 0

    jax.lax.fori_loop(0, n_steps // _B, batch, 0)

    h_last = h_ref[pl.ds(n_nodes - 1, 1), :]
    logits = jnp.dot(h_last, out_w_t_ref[...],
                     preferred_element_type=jnp.float32) + out_b_ref[...]
    lm = logits - jnp.max(logits)
    out_ref[...] = lm - jnp.log(jnp.sum(jnp.exp(lm)))


def _batch_conflict_flags(seqs):
    nb = seqs.shape[1] // _B
    nid = seqs[0].reshape(nb, _B)
    par = seqs[1].reshape(nb, _B)
    pri = seqs[2].reshape(nb, _B)
    wr = nid[:, :, None]
    rd = (wr == par[:, None, :]) | (wr == pri[:, None, :])
    order = jnp.tril(jnp.ones((_B, _B), jnp.bool_), -1).T
    return jnp.any(rd & order[None], axis=(1, 2)).astype(jnp.int32)


def _rnn_scan(seqs, flags, node_emb, W_ih, W_hh, b_ih, b_hh, weight,
              weight_proj, out_W, out_b, *, interpret=False):
    n_nodes, hid = seqs.shape[1], weight.shape[0]
    nclass = out_W.shape[0]
    grid_spec = pltpu.PrefetchScalarGridSpec(
        num_scalar_prefetch=2,
        grid=(1,),
        in_specs=[
            pl.BlockSpec(node_emb.shape, lambda i, s, f: (0, 0)),
            pl.BlockSpec((hid, 3 * hid), lambda i, s, f: (0, 0)),
            pl.BlockSpec((1, 3 * hid), lambda i, s, f: (0, 0)),
            pl.BlockSpec((hid, 3 * hid), lambda i, s, f: (0, 0)),
            pl.BlockSpec((1, 3 * hid), lambda i, s, f: (0, 0)),
            pl.BlockSpec((hid, hid), lambda i, s, f: (0, 0)),
            pl.BlockSpec((1, hid), lambda i, s, f: (0, 0)),
            pl.BlockSpec((hid, nclass), lambda i, s, f: (0, 0)),
            pl.BlockSpec((1, nclass), lambda i, s, f: (0, 0)),
        ],
        out_specs=pl.BlockSpec((1, nclass), lambda i, s, f: (0, 0)),
        scratch_shapes=[pltpu.VMEM((n_nodes, hid), jnp.float32)],
    )
    return pl.pallas_call(
        _scan_body,
        grid_spec=grid_spec,
        out_shape=jax.ShapeDtypeStruct((1, nclass), jnp.float32),
        interpret=interpret,
    )(seqs, flags, node_emb, W_ih.T, b_ih.reshape(1, -1), W_hh.T,
      b_hh.reshape(1, -1), weight, weight_proj.reshape(1, -1),
      out_W.T, out_b.reshape(1, -1))


def kernel(x_index, sequences, embed, weight, weight_proj, W_ih, W_hh, b_ih,
           b_hh, out_W, out_b):
    node_emb = _embedding_mean(x_index, embed)
    seqs = sequences[:, :, 0].T
    flags = _batch_conflict_flags(seqs)
    return _rnn_scan(seqs, flags, node_emb, W_ih, W_hh, b_ih, b_hh, weight,
                     weight_proj, out_W, out_b)
